# baseline (speedup 1.0000x reference)
import jax
import jax.numpy as jnp
from jax.experimental import pallas as pl
from jax.experimental.pallas import tpu as pltpu

T = 2048
D = 1024
FF = 4096
E = 8
TOPK = 2

BLK = 256
MT = T // BLK
FFC = 2048
F = FF // FFC
EP = 128

_INTERPRET = False


def _cumsum_rows(a):
    n = a.shape[0]
    s = 1
    while s < n:
        a = a + jnp.concatenate(
            [jnp.zeros((s, a.shape[1]), a.dtype), a[:-s, :]], axis=0)
        s *= 2
    return a


def _router_body(x_ref, gw_ref, gb_ref, meta_ref, pos_ref, tw_ref):
    x = x_ref[:]
    lg = jax.lax.dot_general(x.astype(jnp.bfloat16),
                             gw_ref[:].astype(jnp.bfloat16),
                             (((1,), (1,)), ((), ())),
                             preferred_element_type=jnp.float32)
    lg = lg + gb_ref[:]
    eidx = jax.lax.broadcasted_iota(jnp.int32, (T, EP), 1)
    m1 = jnp.max(lg, axis=1, keepdims=True)
    i1 = jnp.min(jnp.where(lg == m1, eidx, EP), axis=1, keepdims=True)
    lg2 = jnp.where(eidx == i1, -jnp.inf, lg)
    m2 = jnp.max(lg2, axis=1, keepdims=True)
    i2 = jnp.min(jnp.where(lg2 == m2, eidx, EP), axis=1, keepdims=True)
    den = jnp.sum(jnp.exp(lg - m1), axis=1, keepdims=True)
    wa = 1.0 / den
    wb = jnp.exp(m2 - m1) / den
    tw_ref[:] = jnp.concatenate([wa, wb], axis=1)

    iota_e = jax.lax.broadcasted_iota(jnp.int32, (T, E), 1)
    oh0 = (i1 == iota_e).astype(jnp.float32)
    oh1 = (i2 == iota_e).astype(jnp.float32)
    c0 = _cumsum_rows(oh0)
    c1 = _cumsum_rows(oh1)
    count0 = c0[T - 1:T, :]
    counts = count0 + c1[T - 1:T, :]
    rank0 = jnp.sum(oh0 * (c0 - 1.0), axis=1, keepdims=True)
    rank1 = jnp.sum(oh1 * (count0 + c1 - 1.0), axis=1, keepdims=True)

    pos0 = i1.astype(jnp.float32) * float(T) + rank0
    pos1 = i2.astype(jnp.float32) * float(T) + rank1
    pos_ref[:] = jnp.concatenate([pos0, pos1], axis=1).astype(jnp.int32)

    lane = jax.lax.broadcasted_iota(jnp.int32, (1, EP), 1)
    spread = (jax.lax.broadcasted_iota(jnp.int32, (E, EP), 0)
              == jax.lax.broadcasted_iota(jnp.int32, (E, EP), 1)
              ).astype(jnp.float32)
    counts128 = jax.lax.dot_general(counts, spread, (((1,), (0,)), ((), ())),
                                    precision=jax.lax.Precision.HIGHEST)
    nt = jnp.ceil(counts128 / BLK)
    meta_ref[:] = jnp.where(lane < E, nt, 0.0).astype(jnp.int32)


def _moe_body(meta_ref, x_ref, pos_ref, tw_ref, w1_ref, w3_ref, w2_ref,
              out_ref, xg_ref, acc_ref):
    e = pl.program_id(0)
    f = pl.program_id(1)
    mt = pl.program_id(2)

    @pl.when(jnp.logical_and(e == 0, jnp.logical_and(f == 0, mt == 0)))
    def _():
        out_ref[:] = jnp.zeros_like(out_ref)

    active = mt < meta_ref[e]
    base = e * T + mt * BLK

    @pl.when(jnp.logical_and(active, f == 0))
    def _():
        rid = jax.lax.broadcasted_iota(jnp.int32, (T, BLK), 1) + base
        st = ((pos_ref[:, 0:1] == rid) | (pos_ref[:, 1:2] == rid)
              ).astype(jnp.bfloat16)
        xg_ref[pl.ds(mt * BLK, BLK), :] = jax.lax.dot_general(
            st, x_ref[:], (((0,), (0,)), ((), ())),
            preferred_element_type=jnp.float32).astype(jnp.bfloat16)

    @pl.when(active)
    def _():
        xg = xg_ref[pl.ds(mt * BLK, BLK), :]
        h1 = jax.lax.dot_general(xg, w1_ref[0], (((1,), (1,)), ((), ())),
                                 preferred_element_type=jnp.float32)
        h3 = jax.lax.dot_general(xg, w3_ref[0], (((1,), (1,)), ((), ())),
                                 preferred_element_type=jnp.float32)
        h = h1 * jax.nn.sigmoid(h1) * h3
        part = jax.lax.dot_general(h.astype(jnp.bfloat16), w2_ref[0],
                                   (((1,), (1,)), ((), ())),
                                   preferred_element_type=jnp.float32)

        @pl.when(f == 0)
        def _():
            acc_ref[pl.ds(mt * BLK, BLK), :] = part

        @pl.when(f > 0)
        def _():
            acc_ref[pl.ds(mt * BLK, BLK), :] = (
                acc_ref[pl.ds(mt * BLK, BLK), :] + part)

        @pl.when(f == F - 1)
        def _():
            rid = jax.lax.broadcasted_iota(jnp.int32, (T, BLK), 1) + base
            m0 = pos_ref[:, 0:1] == rid
            m1 = pos_ref[:, 1:2] == rid
            swt = (jnp.where(m0, tw_ref[:, 0:1], 0.0)
                   + jnp.where(m1, tw_ref[:, 1:2], 0.0)).astype(jnp.bfloat16)
            out_ref[:] = out_ref[:] + jax.lax.dot_general(
                swt, acc_ref[pl.ds(mt * BLK, BLK), :].astype(jnp.bfloat16),
                (((1,), (0,)), ((), ())),
                preferred_element_type=jnp.float32)


def kernel(hidden_states, gate_w, gate_b, w1, w2, w3):
    x = hidden_states
    gwp = jnp.zeros((EP, D), jnp.float32).at[:E].set(gate_w)
    gbp = jnp.full((1, EP), -1e30, jnp.float32).at[0, :E].set(gate_b)

    meta, pos, tw = pl.pallas_call(
        _router_body,
        out_shape=(
            jax.ShapeDtypeStruct((1, EP), jnp.int32),
            jax.ShapeDtypeStruct((T, TOPK), jnp.int32),
            jax.ShapeDtypeStruct((T, TOPK), jnp.float32),
        ),
        interpret=_INTERPRET,
    )(x, gwp, gbp)
    meta = meta.reshape(EP)

    grid_spec = pltpu.PrefetchScalarGridSpec(
        num_scalar_prefetch=1,
        grid=(E, F, MT),
        in_specs=[
            pl.BlockSpec((T, D), lambda e, f, mt, m: (0, 0)),
            pl.BlockSpec((T, TOPK), lambda e, f, mt, m: (0, 0)),
            pl.BlockSpec((T, TOPK), lambda e, f, mt, m: (0, 0)),
            pl.BlockSpec((1, FFC, D), lambda e, f, mt, m: (e, f, 0)),
            pl.BlockSpec((1, FFC, D), lambda e, f, mt, m: (e, f, 0)),
            pl.BlockSpec((1, D, FFC), lambda e, f, mt, m: (e, 0, f)),
        ],
        out_specs=pl.BlockSpec((T, D), lambda e, f, mt, m: (0, 0)),
        scratch_shapes=[
            pltpu.VMEM((T, D), jnp.bfloat16),
            pltpu.VMEM((T, D), jnp.float32),
        ],
    )
    out = pl.pallas_call(
        _moe_body,
        grid_spec=grid_spec,
        out_shape=jax.ShapeDtypeStruct((T, D), jnp.float32),
        compiler_params=pltpu.CompilerParams(
            dimension_semantics=("arbitrary", "arbitrary", "arbitrary")),
        interpret=_INTERPRET,
    )(meta, x.astype(jnp.bfloat16), pos, tw,
      w1.astype(jnp.bfloat16), w3.astype(jnp.bfloat16), w2.astype(jnp.bfloat16))
    return out


# speedup vs baseline: 1.5205x; 1.5205x over previous
import jax
import jax.numpy as jnp
from jax.experimental import pallas as pl
from jax.experimental.pallas import tpu as pltpu

T = 2048
D = 1024
FF = 4096
E = 8
TOPK = 2

BLK = 256
G_MAX = (T * TOPK) // BLK + E
FFC = 1024
F = FF // FFC
EP = 128
_ML = 60

_INTERPRET = False


def _cumsum_rows(a):
    n = a.shape[0]
    s = 1
    while s < n:
        a = a + jnp.concatenate(
            [jnp.zeros((s, a.shape[1]), a.dtype), a[:-s, :]], axis=0)
        s *= 2
    return a


def _router_body(x_ref, gw_ref, gb_ref, meta_ref, pos_ref, tw_ref):
    x = x_ref[:]
    lg = jax.lax.dot_general(x.astype(jnp.bfloat16),
                             gw_ref[:].astype(jnp.bfloat16),
                             (((1,), (1,)), ((), ())),
                             preferred_element_type=jnp.float32)
    lg = lg + gb_ref[:]
    eidx = jax.lax.broadcasted_iota(jnp.int32, (T, EP), 1)
    m1 = jnp.max(lg, axis=1, keepdims=True)
    i1 = jnp.min(jnp.where(lg == m1, eidx, EP), axis=1, keepdims=True)
    lg2 = jnp.where(eidx == i1, -jnp.inf, lg)
    m2 = jnp.max(lg2, axis=1, keepdims=True)
    i2 = jnp.min(jnp.where(lg2 == m2, eidx, EP), axis=1, keepdims=True)
    den = jnp.sum(jnp.exp(lg - m1), axis=1, keepdims=True)
    wa = 1.0 / den
    wb = jnp.exp(m2 - m1) / den
    tw_ref[:] = jnp.concatenate([wa, wb], axis=1)

    iota_e = jax.lax.broadcasted_iota(jnp.int32, (T, E), 1)
    oh0 = (i1 == iota_e).astype(jnp.float32)
    oh1 = (i2 == iota_e).astype(jnp.float32)
    c0 = _cumsum_rows(oh0)
    c1 = _cumsum_rows(oh1)
    count0 = c0[T - 1:T, :]
    counts = count0 + c1[T - 1:T, :]
    rank0 = jnp.sum(oh0 * (c0 - 1.0), axis=1, keepdims=True)
    rank1 = jnp.sum(oh1 * (count0 + c1 - 1.0), axis=1, keepdims=True)

    pos0 = i1.astype(jnp.float32) * float(T) + rank0
    pos1 = i2.astype(jnp.float32) * float(T) + rank1
    pos_ref[:] = jnp.concatenate([pos0, pos1], axis=1).astype(jnp.int32)

    ri = jax.lax.broadcasted_iota(jnp.int32, (E, E), 0)
    ci = jax.lax.broadcasted_iota(jnp.int32, (E, E), 1)
    eye = (ri == ci).astype(jnp.float32)
    counts_s = jax.lax.dot_general(eye, counts, (((1,), (1,)), ((), ())),
                                   precision=jax.lax.Precision.HIGHEST)
    nt_s = jnp.ceil(counts_s / BLK)
    tri_s = (ci < ri).astype(jnp.float32)
    cpo_s = jax.lax.dot_general(tri_s, nt_s, (((1,), (0,)), ((), ())),
                                precision=jax.lax.Precision.HIGHEST)
    num_active = jnp.sum(nt_s)
    gi = jax.lax.broadcasted_iota(jnp.int32, (E, EP), 1).astype(jnp.float32)
    te = jnp.sum((cpo_s <= gi).astype(jnp.float32), axis=0, keepdims=True)
    te = jnp.clip(te - 1.0, 0.0, float(E - 1))
    base_g = jnp.max(jnp.where(cpo_s <= gi, cpo_s, 0.0), axis=0,
                     keepdims=True)
    gs = gi - 32.0
    base_s = jnp.max(jnp.where(cpo_s <= gs, cpo_s, 0.0), axis=0,
                     keepdims=True)
    gl = jax.lax.broadcasted_iota(jnp.int32, (1, EP), 1).astype(jnp.float32)
    mt_s = jnp.clip(gl - 32.0 - base_s, 0.0, float(T // BLK - 1))
    lane = jax.lax.broadcasted_iota(jnp.int32, (1, EP), 1)
    meta = jnp.where(lane < G_MAX, te, 0.0)
    meta = jnp.where(jnp.logical_and(lane >= 32, lane < 32 + G_MAX),
                     mt_s, meta)
    meta = jnp.where(lane == _ML, num_active, meta)
    meta_ref[:] = meta.astype(jnp.int32)
    del base_g


def _moe_body(meta_ref, x_ref, pos_ref, tw_ref, w1_ref, w3_ref, w2_ref,
              out_ref, xg_ref, acc_ref, swt_ref):
    g = pl.program_id(0)
    f = pl.program_id(1)

    @pl.when(jnp.logical_and(g == 0, f == 0))
    def _():
        out_ref[:] = jnp.zeros_like(out_ref)

    active = g < meta_ref[_ML]
    base = meta_ref[g] * T + meta_ref[32 + g] * BLK

    @pl.when(jnp.logical_and(active, f == 0))
    def _():
        rid = jax.lax.broadcasted_iota(jnp.int32, (T, BLK), 1) + base
        m0 = pos_ref[:, 0:1] == rid
        m1 = pos_ref[:, 1:2] == rid
        st = (m0 | m1).astype(jnp.float32)
        swt_ref[:] = (jnp.where(m0, tw_ref[:, 0:1], 0.0)
                      + jnp.where(m1, tw_ref[:, 1:2], 0.0))
        xg_ref[:] = jax.lax.dot_general(
            st, x_ref[:], (((0,), (0,)), ((), ())),
            preferred_element_type=jnp.float32)

    @pl.when(active)
    def _():
        xg = xg_ref[:]
        h1 = jax.lax.dot_general(xg, w1_ref[0], (((1,), (1,)), ((), ())),
                                 preferred_element_type=jnp.float32)
        h3 = jax.lax.dot_general(xg, w3_ref[0], (((1,), (1,)), ((), ())),
                                 preferred_element_type=jnp.float32)
        h = h1 * jax.nn.sigmoid(h1) * h3
        part = jax.lax.dot_general(h, w2_ref[0], (((1,), (1,)), ((), ())),
                                   preferred_element_type=jnp.float32)

        @pl.when(f == 0)
        def _():
            acc_ref[:] = part

        @pl.when(f > 0)
        def _():
            acc_ref[:] = acc_ref[:] + part

        @pl.when(f == F - 1)
        def _():
            out_ref[:] = out_ref[:] + jax.lax.dot_general(
                swt_ref[:], acc_ref[:], (((1,), (0,)), ((), ())),
                preferred_element_type=jnp.float32)


def kernel(hidden_states, gate_w, gate_b, w1, w2, w3):
    x = hidden_states
    gwp = jnp.zeros((EP, D), jnp.float32).at[:E].set(gate_w)
    gbp = jnp.full((1, EP), -1e30, jnp.float32).at[0, :E].set(gate_b)

    meta, pos, tw = pl.pallas_call(
        _router_body,
        out_shape=(
            jax.ShapeDtypeStruct((1, EP), jnp.int32),
            jax.ShapeDtypeStruct((T, TOPK), jnp.int32),
            jax.ShapeDtypeStruct((T, TOPK), jnp.float32),
        ),
        interpret=_INTERPRET,
    )(x, gwp, gbp)
    meta = meta.reshape(EP)

    grid_spec = pltpu.PrefetchScalarGridSpec(
        num_scalar_prefetch=1,
        grid=(G_MAX, F),
        in_specs=[
            pl.BlockSpec((T, D), lambda g, f, m: (0, 0)),
            pl.BlockSpec((T, TOPK), lambda g, f, m: (0, 0)),
            pl.BlockSpec((T, TOPK), lambda g, f, m: (0, 0)),
            pl.BlockSpec((1, FFC, D), lambda g, f, m: (
                m[jnp.minimum(g, m[_ML] - 1)],
                jnp.where(g < m[_ML], f, F - 1), 0)),
            pl.BlockSpec((1, FFC, D), lambda g, f, m: (
                m[jnp.minimum(g, m[_ML] - 1)],
                jnp.where(g < m[_ML], f, F - 1), 0)),
            pl.BlockSpec((1, D, FFC), lambda g, f, m: (
                m[jnp.minimum(g, m[_ML] - 1)], 0,
                jnp.where(g < m[_ML], f, F - 1))),
        ],
        out_specs=pl.BlockSpec((T, D), lambda g, f, m: (0, 0)),
        scratch_shapes=[
            pltpu.VMEM((BLK, D), jnp.float32),
            pltpu.VMEM((BLK, D), jnp.float32),
            pltpu.VMEM((T, BLK), jnp.float32),
        ],
    )
    out = pl.pallas_call(
        _moe_body,
        grid_spec=grid_spec,
        out_shape=jax.ShapeDtypeStruct((T, D), jnp.float32),
        compiler_params=pltpu.CompilerParams(
            dimension_semantics=("arbitrary", "arbitrary")),
        interpret=_INTERPRET,
    )(meta, x, pos, tw, w1, w3, w2)
    return out


# speedup vs baseline: 1.5717x; 1.0337x over previous
import jax
import jax.numpy as jnp
from jax.experimental import pallas as pl
from jax.experimental.pallas import tpu as pltpu

T = 2048
D = 1024
FF = 4096
E = 8
TOPK = 2

BLK = 256
G_MAX = (T * TOPK) // BLK + E
FFC = 1024
F = FF // FFC
EP = 128
_ML = 60

_INTERPRET = False


def _cumsum_rows(a):
    n = a.shape[0]
    s = 1
    while s < n:
        a = a + jnp.concatenate(
            [jnp.zeros((s, a.shape[1]), a.dtype), a[:-s, :]], axis=0)
        s *= 2
    return a


def _router_body(x_ref, gw_ref, gb_ref, meta_ref, pos_ref, tw_ref):
    x = x_ref[:]
    lg = jax.lax.dot_general(x.astype(jnp.bfloat16),
                             gw_ref[:].astype(jnp.bfloat16),
                             (((1,), (1,)), ((), ())),
                             preferred_element_type=jnp.float32)
    lg = lg + gb_ref[:]
    eidx = jax.lax.broadcasted_iota(jnp.int32, (T, EP), 1)
    m1 = jnp.max(lg, axis=1, keepdims=True)
    i1 = jnp.min(jnp.where(lg == m1, eidx, EP), axis=1, keepdims=True)
    lg2 = jnp.where(eidx == i1, -jnp.inf, lg)
    m2 = jnp.max(lg2, axis=1, keepdims=True)
    i2 = jnp.min(jnp.where(lg2 == m2, eidx, EP), axis=1, keepdims=True)
    den = jnp.sum(jnp.exp(lg - m1), axis=1, keepdims=True)
    wa = 1.0 / den
    wb = jnp.exp(m2 - m1) / den
    tw_ref[:] = jnp.concatenate([wa, wb], axis=1)

    iota_e = jax.lax.broadcasted_iota(jnp.int32, (T, E), 1)
    oh0 = (i1 == iota_e).astype(jnp.float32)
    oh1 = (i2 == iota_e).astype(jnp.float32)
    c0 = _cumsum_rows(oh0)
    c1 = _cumsum_rows(oh1)
    count0 = c0[T - 1:T, :]
    counts = count0 + c1[T - 1:T, :]
    rank0 = jnp.sum(oh0 * (c0 - 1.0), axis=1, keepdims=True)
    rank1 = jnp.sum(oh1 * (count0 + c1 - 1.0), axis=1, keepdims=True)

    pos0 = i1.astype(jnp.float32) * float(T) + rank0
    pos1 = i2.astype(jnp.float32) * float(T) + rank1
    pos_ref[:] = jnp.concatenate([pos0, pos1], axis=1).astype(jnp.int32)

    ri = jax.lax.broadcasted_iota(jnp.int32, (E, E), 0)
    ci = jax.lax.broadcasted_iota(jnp.int32, (E, E), 1)
    eye = (ri == ci).astype(jnp.float32)
    counts_s = jax.lax.dot_general(eye, counts, (((1,), (1,)), ((), ())),
                                   precision=jax.lax.Precision.HIGHEST)
    nt_s = jnp.ceil(counts_s / BLK)
    tri_s = (ci < ri).astype(jnp.float32)
    cpo_s = jax.lax.dot_general(tri_s, nt_s, (((1,), (0,)), ((), ())),
                                precision=jax.lax.Precision.HIGHEST)
    num_active = jnp.sum(nt_s)
    gi = jax.lax.broadcasted_iota(jnp.int32, (E, EP), 1).astype(jnp.float32)
    te = jnp.sum((cpo_s <= gi).astype(jnp.float32), axis=0, keepdims=True)
    te = jnp.clip(te - 1.0, 0.0, float(E - 1))
    base_g = jnp.max(jnp.where(cpo_s <= gi, cpo_s, 0.0), axis=0,
                     keepdims=True)
    gs = gi - 32.0
    base_s = jnp.max(jnp.where(cpo_s <= gs, cpo_s, 0.0), axis=0,
                     keepdims=True)
    gl = jax.lax.broadcasted_iota(jnp.int32, (1, EP), 1).astype(jnp.float32)
    mt_s = jnp.clip(gl - 32.0 - base_s, 0.0, float(T // BLK - 1))
    lane = jax.lax.broadcasted_iota(jnp.int32, (1, EP), 1)
    meta = jnp.where(lane < G_MAX, te, 0.0)
    meta = jnp.where(jnp.logical_and(lane >= 32, lane < 32 + G_MAX),
                     mt_s, meta)
    meta = jnp.where(lane == _ML, num_active, meta)
    meta_ref[:] = meta.astype(jnp.int32)
    del base_g


def _serp(g, f, m):
    ga = jnp.minimum(g, m[_ML] - 1)
    fa = jnp.where(g < m[_ML], f, F - 1)
    return jnp.where(ga % 2 == 0, fa, F - 1 - fa)


def _moe_body(meta_ref, x_ref, pos_ref, tw_ref, w1_ref, w3_ref, w2_ref,
              out_ref, xg_ref, acc_ref, swt_ref):
    g = pl.program_id(0)
    f = pl.program_id(1)

    @pl.when(jnp.logical_and(g == 0, f == 0))
    def _():
        out_ref[:] = jnp.zeros_like(out_ref)

    active = g < meta_ref[_ML]
    base = meta_ref[g] * T + meta_ref[32 + g] * BLK

    @pl.when(jnp.logical_and(active, f == 0))
    def _():
        rid = jax.lax.broadcasted_iota(jnp.int32, (T, BLK), 1) + base
        m0 = pos_ref[:, 0:1] == rid
        m1 = pos_ref[:, 1:2] == rid
        st = (m0 | m1).astype(jnp.float32)
        swt_ref[:] = (jnp.where(m0, tw_ref[:, 0:1], 0.0)
                      + jnp.where(m1, tw_ref[:, 1:2], 0.0))
        xg_ref[:] = jax.lax.dot_general(
            st, x_ref[:], (((0,), (0,)), ((), ())),
            preferred_element_type=jnp.float32)

    @pl.when(active)
    def _():
        xg = xg_ref[:]
        h1 = jax.lax.dot_general(xg, w1_ref[0], (((1,), (1,)), ((), ())),
                                 preferred_element_type=jnp.float32)
        h3 = jax.lax.dot_general(xg, w3_ref[0], (((1,), (1,)), ((), ())),
                                 preferred_element_type=jnp.float32)
        h = h1 * jax.nn.sigmoid(h1) * h3
        part = jax.lax.dot_general(h, w2_ref[0], (((1,), (1,)), ((), ())),
                                   preferred_element_type=jnp.float32)

        @pl.when(f == 0)
        def _():
            acc_ref[:] = part

        @pl.when(f > 0)
        def _():
            acc_ref[:] = acc_ref[:] + part

        @pl.when(f == F - 1)
        def _():
            out_ref[:] = out_ref[:] + jax.lax.dot_general(
                swt_ref[:], acc_ref[:], (((1,), (0,)), ((), ())),
                preferred_element_type=jnp.float32)


def kernel(hidden_states, gate_w, gate_b, w1, w2, w3):
    x = hidden_states
    gwp = jnp.zeros((EP, D), jnp.float32).at[:E].set(gate_w)
    gbp = jnp.full((1, EP), -1e30, jnp.float32).at[0, :E].set(gate_b)

    meta, pos, tw = pl.pallas_call(
        _router_body,
        out_shape=(
            jax.ShapeDtypeStruct((1, EP), jnp.int32),
            jax.ShapeDtypeStruct((T, TOPK), jnp.int32),
            jax.ShapeDtypeStruct((T, TOPK), jnp.float32),
        ),
        interpret=_INTERPRET,
    )(x, gwp, gbp)
    meta = meta.reshape(EP)

    grid_spec = pltpu.PrefetchScalarGridSpec(
        num_scalar_prefetch=1,
        grid=(G_MAX, F),
        in_specs=[
            pl.BlockSpec((T, D), lambda g, f, m: (0, 0)),
            pl.BlockSpec((T, TOPK), lambda g, f, m: (0, 0)),
            pl.BlockSpec((T, TOPK), lambda g, f, m: (0, 0)),
            pl.BlockSpec((1, FFC, D), lambda g, f, m: (
                m[jnp.minimum(g, m[_ML] - 1)], _serp(g, f, m), 0)),
            pl.BlockSpec((1, FFC, D), lambda g, f, m: (
                m[jnp.minimum(g, m[_ML] - 1)], _serp(g, f, m), 0)),
            pl.BlockSpec((1, D, FFC), lambda g, f, m: (
                m[jnp.minimum(g, m[_ML] - 1)], 0, _serp(g, f, m))),
        ],
        out_specs=pl.BlockSpec((T, D), lambda g, f, m: (0, 0)),
        scratch_shapes=[
            pltpu.VMEM((BLK, D), jnp.float32),
            pltpu.VMEM((BLK, D), jnp.float32),
            pltpu.VMEM((T, BLK), jnp.float32),
        ],
    )
    out = pl.pallas_call(
        _moe_body,
        grid_spec=grid_spec,
        out_shape=jax.ShapeDtypeStruct((T, D), jnp.float32),
        compiler_params=pltpu.CompilerParams(
            dimension_semantics=("arbitrary", "arbitrary")),
        interpret=_INTERPRET,
    )(meta, x, pos, tw, w1, w3, w2)
    return out
